# baseline (speedup 1.0000x reference)
import functools
import jax
import jax.numpy as jnp
from jax.experimental import pallas as pl
from jax.experimental.pallas import tpu as pltpu

_LANES = 128


def _relu(x):
    return jnp.maximum(x, 0.0)


def _a1_body(bnd_ref, s_ref, d_ref, xc_ref, out_ref):
    i = pl.program_id(0)
    blk = xc_ref.shape[0]
    base = i * blk

    @pl.when(i == 0)
    def _():
        out_ref[...] = jnp.zeros_like(out_ref)

    lane = jax.lax.broadcasted_iota(jnp.int32, (1, 2), 1)
    lo = bnd_ref[i, 0]
    hi = bnd_ref[i + 1, 0]

    def chunk(c, _):
        svec = s_ref[pl.ds(c, 1), :]
        dvec = d_ref[pl.ds(c, 1), :]
        for j in range(_LANES):
            s = svec[0, j] - base
            d = dvec[0, j]
            valid = (s >= 0) & (s < blk)
            se = jnp.where(valid, s, 0)
            w = jnp.where(valid, 1.0, 0.0)
            xv = xc_ref[pl.ds(se, 1), :]
            out_ref[pl.ds(d, 1), :] += jnp.where(lane == 0, xv[0, 0], 1.0) * w
        return 0

    jax.lax.fori_loop(lo // _LANES, (hi + _LANES - 1) // _LANES, chunk, 0)


def _a2_body(s_ref, d_ref, xm_ref, out_ref):
    out_ref[...] = jnp.zeros_like(out_ref)
    lane = jax.lax.broadcasted_iota(jnp.int32, (1, 2), 1)

    def chunk(c, _):
        svec = s_ref[pl.ds(c, 1), :]
        dvec = d_ref[pl.ds(c, 1), :]
        for j in range(_LANES):
            s = svec[0, j]
            d = dvec[0, j]
            xv = xm_ref[pl.ds(s, 1), :]
            out_ref[pl.ds(d, 1), :] += jnp.where(lane == 0, xv[0, 0], 1.0)
        return 0

    jax.lax.fori_loop(0, s_ref.shape[0], chunk, 0)


def _e_body(bnd_ref, s_ref, d_ref, xg_ref, tg_ref, xc_ref,
            vc_ref, vg_ref, w2s_ref, w2g_ref, c2_ref, tc_ref,
            a1_ref, cnt_ref, a2_ref):
    i = pl.program_id(0)
    blk = c2_ref.shape[0]
    base = i * blk
    a1_ref[...] = jnp.zeros_like(a1_ref)
    cnt_ref[...] = jnp.zeros_like(cnt_ref)
    a2_ref[...] = jnp.zeros_like(a2_ref)
    lo = bnd_ref[i, 0]
    hi = bnd_ref[i + 1, 0]
    kg = tg_ref.shape[1]
    hid = c2_ref.shape[1]

    def chunk(c, _):
        svec = s_ref[pl.ds(c, 1), :]
        dvec = d_ref[pl.ds(c, 1), :]
        for j in range(_LANES):
            s = svec[0, j]
            d = dvec[0, j] - base
            valid = (d >= 0) & (d < blk)
            de = jnp.where(valid, d, blk)
            a1_ref[pl.ds(de, 1), :] += xg_ref[pl.ds(s, 1), :]
            cnt_ref[pl.ds(de, 1), :] += 1.0
            t = tg_ref[pl.ds(s, 1), :]
            row = jnp.zeros((1, hid), jnp.float32)
            for q in range(kg):
                row = row + t[0, q] * vg_ref[pl.ds(q, 1), :]
            a2_ref[pl.ds(de, 1), :] += _relu(row)
        return 0

    jax.lax.fori_loop(lo // _LANES, (hi + _LANES - 1) // _LANES, chunk, 0)

    cnt = jnp.maximum(cnt_ref[0:blk, :], 1.0)
    tc = jnp.concatenate([xc_ref[...], a1_ref[0:blk, :] / cnt], axis=1)
    tc_ref[...] = tc
    hc = _relu(jnp.dot(tc, vc_ref[...], preferred_element_type=jnp.float32))
    c2_ref[...] = _relu(
        jnp.dot(hc, w2s_ref[...], preferred_element_type=jnp.float32)
        + jnp.dot(a2_ref[0:blk, :] / cnt, w2g_ref[...],
                  preferred_element_type=jnp.float32))


def _f_body(s_ref, d_ref, xg_ref, tg_ref, xm_ref,
            vm_ref, vg_ref, w2s_ref, w2g_ref, m2_ref, tm_ref,
            a1_ref, cnt_ref, a2_ref):
    n = m2_ref.shape[0]
    hid = m2_ref.shape[1]
    a1_ref[...] = jnp.zeros_like(a1_ref)
    cnt_ref[...] = jnp.zeros_like(cnt_ref)
    a2_ref[...] = jnp.zeros_like(a2_ref)
    kg = tg_ref.shape[1]

    def chunk(c, _):
        svec = s_ref[pl.ds(c, 1), :]
        dvec = d_ref[pl.ds(c, 1), :]
        for j in range(_LANES):
            s = svec[0, j]
            d = dvec[0, j]
            a1_ref[pl.ds(d, 1), :] += xg_ref[pl.ds(s, 1), :]
            cnt_ref[pl.ds(d, 1), :] += 1.0
            t = tg_ref[pl.ds(s, 1), :]
            row = jnp.zeros((1, hid), jnp.float32)
            for q in range(kg):
                row = row + t[0, q] * vg_ref[pl.ds(q, 1), :]
            a2_ref[pl.ds(d, 1), :] += _relu(row)
        return 0

    jax.lax.fori_loop(0, s_ref.shape[0], chunk, 0)

    cnt = jnp.maximum(cnt_ref[0:n, :], 1.0)
    tm = jnp.concatenate([xm_ref[...], a1_ref[0:n, :] / cnt], axis=1)
    tm_ref[...] = tm
    hm = _relu(jnp.dot(tm, vm_ref[...], preferred_element_type=jnp.float32))
    m2_ref[...] = _relu(
        jnp.dot(hm, w2s_ref[...], preferred_element_type=jnp.float32)
        + jnp.dot(a2_ref[0:n, :] / cnt, w2g_ref[...],
                  preferred_element_type=jnp.float32))


def _c1_body(bnd_ref, s_ref, d_ref, tc_ref, vc_ref, out_ref):
    i = pl.program_id(0)
    blk = tc_ref.shape[0]
    base = i * blk
    k = tc_ref.shape[1]
    hid = out_ref.shape[1]

    @pl.when(i == 0)
    def _():
        out_ref[...] = jnp.zeros_like(out_ref)

    lo = bnd_ref[i, 0]
    hi = bnd_ref[i + 1, 0]

    def chunk(c, _):
        svec = s_ref[pl.ds(c, 1), :]
        dvec = d_ref[pl.ds(c, 1), :]
        for j in range(_LANES):
            s = svec[0, j] - base
            d = dvec[0, j]
            valid = (s >= 0) & (s < blk)
            se = jnp.where(valid, s, 0)
            w = jnp.where(valid, 1.0, 0.0)
            t = tc_ref[pl.ds(se, 1), :]
            row = jnp.zeros((1, hid), jnp.float32)
            for q in range(k):
                row = row + t[0, q] * vc_ref[pl.ds(q, 1), :]
            out_ref[pl.ds(d, 1), :] += _relu(row) * w
        return 0

    jax.lax.fori_loop(lo // _LANES, (hi + _LANES - 1) // _LANES, chunk, 0)


def _c2s_body(s_ref, d_ref, tm_ref, vm_ref, out_ref):
    out_ref[...] = jnp.zeros_like(out_ref)
    k = tm_ref.shape[1]
    hid = out_ref.shape[1]

    def chunk(c, _):
        svec = s_ref[pl.ds(c, 1), :]
        dvec = d_ref[pl.ds(c, 1), :]
        for j in range(_LANES):
            s = svec[0, j]
            d = dvec[0, j]
            t = tm_ref[pl.ds(s, 1), :]
            row = jnp.zeros((1, hid), jnp.float32)
            for q in range(k):
                row = row + t[0, q] * vm_ref[pl.ds(q, 1), :]
            out_ref[pl.ds(d, 1), :] += _relu(row)
        return 0

    jax.lax.fori_loop(0, s_ref.shape[0], chunk, 0)


def _g2_body(tg_ref, vg_ref, scg_ref, ccg_ref, smg_ref, cmg_ref,
             w2s_ref, w2c_ref, w2m_ref, out_ref):
    hg = _relu(jnp.dot(tg_ref[...], vg_ref[...],
                       preferred_element_type=jnp.float32))
    acg = scg_ref[...] / jnp.maximum(ccg_ref[...], 1.0)
    amg = smg_ref[...] / jnp.maximum(cmg_ref[...], 1.0)
    out_ref[...] = _relu(
        jnp.dot(hg, w2s_ref[...], preferred_element_type=jnp.float32)
        + jnp.dot(acg, w2c_ref[...], preferred_element_type=jnp.float32)
        + jnp.dot(amg, w2m_ref[...], preferred_element_type=jnp.float32))


def _pool_body(b_ref, x_ref, sum_ref, cnt_ref, nb):
    @pl.when(pl.program_id(0) == 0)
    def _():
        sum_ref[...] = jnp.zeros_like(sum_ref)
        cnt_ref[...] = jnp.zeros_like(cnt_ref)

    seg = b_ref[...]
    ids = jax.lax.broadcasted_iota(jnp.int32, (nb, seg.shape[0]), 0)
    mask = (ids == seg[:, 0][None, :]).astype(jnp.float32)
    sum_ref[...] += jnp.dot(mask, x_ref[...],
                            preferred_element_type=jnp.float32)
    cnt_ref[...] += jnp.sum(mask, axis=1, keepdims=True)


def _heads_body(sg_ref, cg_ref, sc_ref, cc_ref, sm_ref, cm_ref,
                pg_ref, pc_ref, pm_ref,
                mg_ref, mc_ref, mm_ref, f_ref):
    mg = jnp.dot(sg_ref[...] / jnp.maximum(cg_ref[...], 1.0), pg_ref[...],
                 preferred_element_type=jnp.float32)
    mc = jnp.dot(sc_ref[...] / jnp.maximum(cc_ref[...], 1.0), pc_ref[...],
                 preferred_element_type=jnp.float32)
    mm = jnp.dot(sm_ref[...] / jnp.maximum(cm_ref[...], 1.0), pm_ref[...],
                 preferred_element_type=jnp.float32)
    mg_ref[...] = mg
    mc_ref[...] = mc
    mm_ref[...] = mm
    f_ref[...] = (mg + mc + mm) / 3.0


def _i32(a):
    return a.astype(jnp.int32)


def _pad_edges(s, d, dummy):
    e = s.shape[0]
    rows = -(-e // _LANES)
    rows = -(-rows // 8) * 8
    tot = rows * _LANES
    s2 = jnp.concatenate([_i32(s), jnp.zeros((tot - e,), jnp.int32)])
    d2 = jnp.concatenate([_i32(d), jnp.full((tot - e,), dummy, jnp.int32)])
    return s2.reshape(rows, _LANES), d2.reshape(rows, _LANES)


def kernel(x_gene, x_cpg, x_mirna, ei_cpg_gene, ei_mirna_gene, ei_gene_cpg,
           ei_gene_mirna, batch_gene, batch_cpg, batch_mirna,
           W1_self_g, W1_cg, W1_mg, W1_self_c, W1_gc, W1_self_m, W1_gm,
           W2_self_g, W2_cg, W2_mg, W2_self_c, W2_gc, W2_self_m, W2_gm,
           P_g, P_c, P_m):
    n_g, n_c, n_m = x_gene.shape[0], x_cpg.shape[0], x_mirna.shape[0]
    hid = W1_self_g.shape[1]
    nb = 8
    f32 = jnp.float32
    blk_c = 4000
    grid_c = n_c // blk_c

    full = lambda a: pl.BlockSpec(a.shape, lambda i: tuple(0 for _ in a.shape))

    p_gc = jnp.argsort(ei_gene_cpg[1])
    gc_s, gc_d = _pad_edges(ei_gene_cpg[0][p_gc], ei_gene_cpg[1][p_gc], n_c)
    bnd_gc = _i32(jnp.searchsorted(
        _i32(ei_gene_cpg[1][p_gc]), jnp.arange(0, n_c + 1, blk_c))
    ).reshape(-1, 1)

    p_cg = jnp.argsort(ei_cpg_gene[0])
    cg_s, cg_d = _pad_edges(ei_cpg_gene[0][p_cg], ei_cpg_gene[1][p_cg], n_g)
    bnd_cg = _i32(jnp.searchsorted(
        _i32(ei_cpg_gene[0][p_cg]), jnp.arange(0, n_c + 1, blk_c))
    ).reshape(-1, 1)
    mg_s, mg_d = _pad_edges(ei_mirna_gene[0], ei_mirna_gene[1], n_g)
    gm_s, gm_d = _pad_edges(ei_gene_mirna[0], ei_gene_mirna[1], n_m)

    a_cg = pl.pallas_call(
        _a1_body,
        grid=(grid_c,),
        in_specs=[full(bnd_cg), full(cg_s), full(cg_d),
                  pl.BlockSpec((blk_c, 1), lambda i: (i, 0))],
        out_specs=pl.BlockSpec((n_g + 8, 2), lambda i: (0, 0)),
        out_shape=jax.ShapeDtypeStruct((n_g + 8, 2), f32),
    )(bnd_cg, cg_s, cg_d, x_cpg)

    a_mg = pl.pallas_call(
        _a2_body,
        out_shape=jax.ShapeDtypeStruct((n_g + 8, 2), f32),
    )(mg_s, mg_d, x_mirna)

    m_cg = a_cg[:n_g, 0:1] / jnp.maximum(a_cg[:n_g, 1:2], 1.0)
    m_mg = a_mg[:n_g, 0:1] / jnp.maximum(a_mg[:n_g, 1:2], 1.0)
    c_cg = a_cg[:n_g, 1:2]
    c_mg = a_mg[:n_g, 1:2]

    T_g = jnp.concatenate([x_gene, m_cg, m_mg], axis=1)
    V_g = jnp.concatenate([W1_self_g, W1_cg, W1_mg], axis=0)
    V_c = jnp.concatenate([W1_self_c, W1_gc], axis=0)
    V_m = jnp.concatenate([W1_self_m, W1_gm], axis=0)

    rowc = lambda w: pl.BlockSpec((blk_c, w), lambda i: (i, 0))
    c2, T_c = pl.pallas_call(
        _e_body,
        grid=(grid_c,),
        in_specs=[full(bnd_gc), full(gc_s), full(gc_d), full(x_gene),
                  full(T_g), rowc(1), full(V_c), full(V_g),
                  full(W2_self_c), full(W2_gc)],
        out_specs=[rowc(hid), rowc(3)],
        out_shape=[jax.ShapeDtypeStruct((n_c, hid), f32),
                   jax.ShapeDtypeStruct((n_c, 3), f32)],
        scratch_shapes=[pltpu.VMEM((blk_c + 8, 2), f32),
                        pltpu.VMEM((blk_c + 8, 1), f32),
                        pltpu.VMEM((blk_c + 8, hid), f32)],
    )(bnd_gc, gc_s, gc_d, x_gene, T_g, x_cpg, V_c, V_g, W2_self_c, W2_gc)

    m2, T_m = pl.pallas_call(
        _f_body,
        out_shape=[jax.ShapeDtypeStruct((n_m, hid), f32),
                   jax.ShapeDtypeStruct((n_m, 3), f32)],
        scratch_shapes=[pltpu.VMEM((n_m + 8, 2), f32),
                        pltpu.VMEM((n_m + 8, 1), f32),
                        pltpu.VMEM((n_m + 8, hid), f32)],
    )(gm_s, gm_d, x_gene, T_g, x_mirna, V_m, V_g, W2_self_m, W2_gm)

    s2_cg = pl.pallas_call(
        _c1_body,
        grid=(grid_c,),
        in_specs=[full(bnd_cg), full(cg_s), full(cg_d), rowc(3), full(V_c)],
        out_specs=pl.BlockSpec((n_g + 8, hid), lambda i: (0, 0)),
        out_shape=jax.ShapeDtypeStruct((n_g + 8, hid), f32),
    )(bnd_cg, cg_s, cg_d, T_c, V_c)

    s2_mg = pl.pallas_call(
        _c2s_body,
        out_shape=jax.ShapeDtypeStruct((n_g + 8, hid), f32),
    )(mg_s, mg_d, T_m, V_m)

    blk_g = 2000
    rowg = lambda w: pl.BlockSpec((blk_g, w), lambda i: (i, 0))
    g2 = pl.pallas_call(
        _g2_body,
        grid=(n_g // blk_g,),
        in_specs=[rowg(T_g.shape[1]), full(V_g),
                  rowg(hid), rowg(1), rowg(hid), rowg(1),
                  full(W2_self_g), full(W2_cg), full(W2_mg)],
        out_specs=rowg(hid),
        out_shape=jax.ShapeDtypeStruct((n_g, hid), f32),
    )(T_g, V_g, s2_cg[:n_g], c_cg, s2_mg[:n_g], c_mg,
      W2_self_g, W2_cg, W2_mg)

    def pool(batch, x, blk):
        n = x.shape[0]
        return pl.pallas_call(
            functools.partial(_pool_body, nb=nb),
            grid=(n // blk,),
            in_specs=[pl.BlockSpec((blk, 1), lambda i: (i, 0)),
                      pl.BlockSpec((blk, hid), lambda i: (i, 0))],
            out_specs=[pl.BlockSpec((nb, hid), lambda i: (0, 0)),
                       pl.BlockSpec((nb, 1), lambda i: (0, 0))],
            out_shape=[jax.ShapeDtypeStruct((nb, hid), f32),
                       jax.ShapeDtypeStruct((nb, 1), f32)],
        )(_i32(batch).reshape(-1, 1), x)

    sg, cg = pool(batch_gene, g2, 2000)
    sc, cc = pool(batch_cpg, c2, 4000)
    sm, cm = pool(batch_mirna, m2, 2000)

    mod_g, mod_c, mod_m, fused = pl.pallas_call(
        _heads_body,
        out_shape=[jax.ShapeDtypeStruct((nb, hid), f32)] * 4,
    )(sg, cg, sc, cc, sm, cm, P_g, P_c, P_m)

    return (g2, c2, m2, mod_g, mod_c, mod_m, fused)
